# baseline (speedup 1.0000x reference)
import functools

import jax
import jax.numpy as jnp
from jax import lax
from jax.experimental import pallas as pl
from jax.experimental.pallas import tpu as pltpu
from jax.experimental.pallas import tpu_sc as plsc

B = 64
S = 1025
SP = 1040
D = 768
K = 8
NC = 2
NS = 16
L = 16


def _dist_body(x_ref, out_ref):
    g = x_ref[0, pl.ds(0, 1), :]
    gn = jnp.sqrt(jnp.sum(g * g))
    for c in range(8):
        rows = x_ref[0, pl.ds(c * 128, 128), :]
        num = jnp.sum(rows * g, axis=1, keepdims=True)
        ln = jnp.sqrt(jnp.sum(rows * rows, axis=1, keepdims=True))
        dist = num / jnp.maximum(gn * ln, 1e-8)
        if c == 0:
            rid = lax.broadcasted_iota(jnp.int32, (128, 1), 0)
            dist = jnp.where(rid == 0, -jnp.inf, dist)
        out_ref[0, pl.ds(c * 128, 128), :] = dist
    r = x_ref[0, pl.ds(1024, 1), :]
    num = jnp.sum(r * g)
    ln = jnp.sqrt(jnp.sum(r * r))
    d_last = num / jnp.maximum(gn * ln, 1e-8)
    tid = lax.broadcasted_iota(jnp.int32, (16, 1), 0)
    tail = jnp.where(tid == 0, d_last, -jnp.inf)
    out_ref[0, pl.ds(1024, 16), :] = tail


def _distances(x):
    return pl.pallas_call(
        _dist_body,
        grid=(B,),
        in_specs=[pl.BlockSpec((1, S, D), lambda b: (b, 0, 0))],
        out_specs=pl.BlockSpec((1, SP, 1), lambda b: (b, 0, 0)),
        out_shape=jax.ShapeDtypeStruct((B, SP, 1), jnp.float32),
    )(x)


@functools.lru_cache(maxsize=None)
def _topk_gather_kernel():
    @functools.partial(
        pl.kernel,
        out_type=jax.ShapeDtypeStruct((B, K, D), jnp.float32),
        mesh=plsc.VectorSubcoreMesh(core_axis_name="c", subcore_axis_name="s"),
        scratch_types=[
            pltpu.VMEM((SP,), jnp.float32),
            pltpu.VMEM((L,), jnp.int32),
            pltpu.VMEM((L, D), jnp.float32),
            pltpu.SemaphoreType.DMA,
        ],
        compiler_params=pltpu.CompilerParams(needs_layout_passes=False),
    )
    def _topk_gather(dist_hbm, x2d_hbm, out_hbm, dist_v, idx_v, rows_v, sem):
        wid = lax.axis_index("s") * NC + lax.axis_index("c")
        iota = jnp.arange(L, dtype=jnp.int32)
        for i in range(B // (NC * NS)):
            b = wid * (B // (NC * NS)) + i
            pltpu.sync_copy(dist_hbm.at[b], dist_v)
            tv, ti = plsc.sort_key_val(dist_v[pl.ds(0, L)], iota,
                                       descending=True)
            for j in range(1, SP // L):
                sv, si = plsc.sort_key_val(dist_v[pl.ds(j * L, L)],
                                           iota + (j * L), descending=True)
                rv = lax.rev(sv, (0,))
                ri = lax.rev(si, (0,))
                m = tv >= rv
                hi = jnp.where(m, tv, rv)
                hx = jnp.where(m, ti, ri)
                tv, ti = plsc.sort_key_val(hi, hx, descending=True)
            idx_v[...] = ti + b * S
            pltpu.async_copy(x2d_hbm.at[idx_v], rows_v, sem).wait()
            pltpu.sync_copy(rows_v.at[pl.ds(0, K)], out_hbm.at[b])

    return _topk_gather


def kernel(x, images):
    del images
    dist = _distances(x).reshape(B, SP)
    x2d = x.reshape(B * S, D)
    return _topk_gather_kernel()(dist, x2d)


# speedup vs baseline: 1.0352x; 1.0352x over previous
import functools

import jax
import jax.numpy as jnp
from jax import lax
from jax.experimental import pallas as pl
from jax.experimental.pallas import tpu as pltpu
from jax.experimental.pallas import tpu_sc as plsc

B = 64
S = 1025
SP = 1040
D = 768
K = 8
NC = 2
NS = 16
L = 16


_DN_T = (((1,), (1,)), ((), ()))


def _dist_body(x_ref, out_ref):
    g = x_ref[0, pl.ds(0, 1), :]
    gn = jnp.sqrt(jnp.sum(g * g))
    ones_row = jnp.ones((1, D), jnp.float32)
    for c in range(8):
        rows = x_ref[0, pl.ds(c * 128, 128), :]
        num = lax.dot_general(g, rows, _DN_T,
                              preferred_element_type=jnp.float32)[0]
        l2 = lax.dot_general(ones_row, rows * rows, _DN_T,
                             preferred_element_type=jnp.float32)[0]
        ln = jnp.sqrt(l2)
        dist = num / jnp.maximum(gn * ln, 1e-8)
        if c == 0:
            rid = lax.broadcasted_iota(jnp.int32, (128,), 0)
            dist = jnp.where(rid == 0, -jnp.inf, dist)
        out_ref[0, 0, pl.ds(c * 128, 128)] = dist
    r = x_ref[0, pl.ds(1024, 1), :]
    num = jnp.sum(r * g)
    ln = jnp.sqrt(jnp.sum(r * r))
    d_last = num / jnp.maximum(gn * ln, 1e-8)
    tid = lax.broadcasted_iota(jnp.int32, (16,), 0)
    tail = jnp.where(tid == 0, d_last, -jnp.inf)
    out_ref[0, 0, pl.ds(1024, 16)] = tail


def _distances(x):
    return pl.pallas_call(
        _dist_body,
        grid=(B,),
        in_specs=[pl.BlockSpec((1, S, D), lambda b: (b, 0, 0))],
        out_specs=pl.BlockSpec((1, 1, SP), lambda b: (b, 0, 0)),
        out_shape=jax.ShapeDtypeStruct((B, 1, SP), jnp.float32),
    )(x)


@functools.lru_cache(maxsize=None)
def _topk_gather_kernel():
    @functools.partial(
        pl.kernel,
        out_type=jax.ShapeDtypeStruct((B, K, D), jnp.float32),
        mesh=plsc.VectorSubcoreMesh(core_axis_name="c", subcore_axis_name="s"),
        scratch_types=[
            pltpu.VMEM((SP,), jnp.float32),
            pltpu.VMEM((L,), jnp.int32),
            pltpu.VMEM((L, D), jnp.float32),
            pltpu.SemaphoreType.DMA,
        ],
        compiler_params=pltpu.CompilerParams(needs_layout_passes=False),
    )
    def _topk_gather(dist_hbm, x2d_hbm, out_hbm, dist_v, idx_v, rows_v, sem):
        wid = lax.axis_index("s") * NC + lax.axis_index("c")
        iota = jnp.arange(L, dtype=jnp.int32)
        for i in range(B // (NC * NS)):
            b = wid * (B // (NC * NS)) + i
            pltpu.sync_copy(dist_hbm.at[b], dist_v)
            tv, ti = plsc.sort_key_val(dist_v[pl.ds(0, L)], iota,
                                       descending=True)
            for j in range(1, SP // L):
                sv, si = plsc.sort_key_val(dist_v[pl.ds(j * L, L)],
                                           iota + (j * L), descending=True)
                rv = lax.rev(sv, (0,))
                ri = lax.rev(si, (0,))
                m = tv >= rv
                hi = jnp.where(m, tv, rv)
                hx = jnp.where(m, ti, ri)
                tv, ti = plsc.sort_key_val(hi, hx, descending=True)
            idx_v[...] = ti + b * S
            pltpu.async_copy(x2d_hbm.at[idx_v], rows_v, sem).wait()
            pltpu.sync_copy(rows_v.at[pl.ds(0, K)], out_hbm.at[b])

    return _topk_gather


def kernel(x, images):
    del images
    dist = _distances(x).reshape(B, SP)
    x2d = x.reshape(B * S, D)
    return _topk_gather_kernel()(dist, x2d)
